# Initial kernel scaffold; baseline (speedup 1.0000x reference)
import jax
import jax.numpy as jnp
from jax.experimental import pallas as pl


def kernel(inputs, unnormalized_widths, unnormalized_heights, unnormalized_derivatives):
    raise NotImplementedError("write your pallas kernel here")


# speedup vs baseline: 342.2766x; 342.2766x over previous
import functools

import jax
import jax.numpy as jnp
from jax import lax
from jax.experimental import pallas as pl
from jax.experimental.pallas import tpu as pltpu
from jax.experimental.pallas import tpu_sc as plsc

BATCH = 32768
V = 32
NUM_BINS = 30
MIN_BIN_W = 0.001
MIN_BIN_H = 0.001
MIN_DERIV = 0.001
_EDGE_CONST = 0.5392745158594121

NC = 2
NS = 16
NW = NC * NS
ROWS_PER_TILE = BATCH // NW

TBL = 1024
OFF_CB, OFF_CW, OFF_WI, OFF_CH, OFF_H, OFF_D, OFF_DL = (k * TBL for k in range(7))

LN2 = 0.6931471805599453
_LOG2_C = (-0.0251232, 0.27003746, -1.2479625, 3.24946656, -5.30170911,
           6.08989576, -3.03460285)


def _tables_tc_kernel(uw_ref, uh_ref, ud_ref, out_ref):
    uw = uw_ref[...]
    uh = uh_ref[...]
    ud = ud_ref[...]

    bi = lax.broadcasted_iota(jnp.int32, (NUM_BINS, V), 0)
    ji = lax.broadcasted_iota(jnp.int32, (NUM_BINS, V), 1)
    m = (bi < ji).astype(jnp.float32)

    def cumparams(u, min_size):
        mx = jnp.max(u, axis=-1, keepdims=True)
        e = jnp.exp(u - mx)
        p = e / jnp.sum(e, axis=-1, keepdims=True)
        p = min_size + (1.0 - min_size * NUM_BINS) * p
        cum = jnp.dot(p, m, preferred_element_type=jnp.float32,
                      precision=lax.Precision.HIGHEST)
        return cum

    cumw = cumparams(uw, MIN_BIN_W)
    cumh = cumparams(uh, MIN_BIN_H)
    w30 = cumw[:, 1:31] - cumw[:, 0:30]
    h30 = cumh[:, 1:31] - cumh[:, 0:30]
    ones2 = jnp.ones((V, 2), jnp.float32)

    col = lax.broadcasted_iota(jnp.int32, (V, V), 1)
    cb = jnp.where(col == 30, cumw + 1e-6, cumw)
    cb = jnp.where(col == 31, 1e30, cb)

    winv = jnp.concatenate([1.0 / w30, ones2], axis=1)
    hpad = jnp.concatenate([h30, ones2], axis=1)
    dlpad = jnp.concatenate([h30 / w30, ones2], axis=1)

    edge = jnp.full((V, 1), _EDGE_CONST, jnp.float32)
    dp31 = jnp.concatenate([edge, ud, edge], axis=1)
    sp = jnp.maximum(dp31, 0.0) + jnp.log(1.0 + jnp.exp(-jnp.abs(dp31)))
    dpad = jnp.concatenate([MIN_DERIV + sp, jnp.ones((V, 1), jnp.float32)],
                           axis=1)

    out_ref[0] = cb
    out_ref[1] = cumw
    out_ref[2] = winv
    out_ref[3] = cumh
    out_ref[4] = hpad
    out_ref[5] = dpad
    out_ref[6] = dlpad


def _build_tables(uw, uh, ud):
    return pl.pallas_call(
        _tables_tc_kernel,
        out_shape=jax.ShapeDtypeStruct((7, V, V), jnp.float32),
    )(uw, uh, ud)


def _log2_poly(x):
    i = lax.bitcast_convert_type(x, jnp.int32)
    e = lax.convert_element_type(
        lax.shift_right_arithmetic(i, 23) - 127, jnp.float32)
    mb = lax.bitwise_or(lax.bitwise_and(i, 0x007FFFFF), 0x3F800000)
    mm = lax.bitcast_convert_type(mb, jnp.float32)
    p = jnp.full((16,), _LOG2_C[0], jnp.float32)
    for c in _LOG2_C[1:]:
        p = p * mm + c
    return e + p


ELEMS_PER_TILE = ROWS_PER_TILE * V


def _sc_body(x_hbm, tab_hbm, out_hbm, det_hbm, tab_v, xin_v, out_v, det_v):
    wid = lax.axis_index("s") * NC + lax.axis_index("c")
    e0 = wid * ELEMS_PER_TILE
    pltpu.sync_copy(tab_hbm, tab_v)
    pltpu.sync_copy(x_hbm.at[pl.ds(e0, ELEMS_PER_TILE)], xin_v)

    lanes = lax.iota(jnp.int32, 16)
    vb0 = lanes * V
    vb1 = vb0 + 16 * V

    def do_vec(off, vbase):
        x = xin_v[pl.ds(off, 16)]
        b = jnp.zeros((16,), jnp.int32)
        for k in (16, 8, 4, 2, 1):
            t = b + k
            cbv = plsc.load_gather(tab_v, [vbase + t])
            b = jnp.where(cbv <= x, t, b)
        binr = jnp.minimum(b, NUM_BINS - 1)
        gi = vbase + binr
        cw = plsc.load_gather(tab_v, [gi + OFF_CW])
        winv = plsc.load_gather(tab_v, [gi + OFF_WI])
        ch = plsc.load_gather(tab_v, [gi + OFF_CH])
        h = plsc.load_gather(tab_v, [gi + OFF_H])
        d = plsc.load_gather(tab_v, [gi + OFF_D])
        dp = plsc.load_gather(tab_v, [gi + (OFF_D + 1)])
        dl = plsc.load_gather(tab_v, [gi + OFF_DL])

        th = (x - cw) * winv
        th2 = th * th
        th1 = th - th2
        num = h * (dl * th2 + d * th1)
        den = dl + (d + dp - 2.0 * dl) * th1
        spl = ch + num / den
        omt = 1.0 - th
        nd = (dl * dl) * (dp * th2 + 2.0 * dl * th1 + d * (omt * omt))
        logdet = LN2 * (_log2_poly(nd) - 2.0 * _log2_poly(den))
        inside = jnp.logical_and(x >= 0.0, x <= 1.0)
        out_v[pl.ds(off, 16)] = jnp.where(inside, spl, x)
        det_v[pl.ds(off, 16)] = jnp.where(
            inside, logdet, jnp.zeros((16,), jnp.float32))

    def body(i, carry):
        base = i * 64
        do_vec(base, vb0)
        do_vec(base + 16, vb1)
        do_vec(base + 32, vb0)
        do_vec(base + 48, vb1)
        return carry

    lax.fori_loop(0, ELEMS_PER_TILE // 64, body, 0)

    pltpu.sync_copy(out_v, out_hbm.at[pl.ds(e0, ELEMS_PER_TILE)])
    pltpu.sync_copy(det_v, det_hbm.at[pl.ds(e0, ELEMS_PER_TILE)])


@functools.partial(jax.jit)
def _sc_spline(x, tab_flat):
    mesh = plsc.VectorSubcoreMesh(core_axis_name="c", subcore_axis_name="s")
    f = functools.partial(
        pl.kernel,
        out_type=[
            jax.ShapeDtypeStruct((BATCH * V,), jnp.float32),
            jax.ShapeDtypeStruct((BATCH * V,), jnp.float32),
        ],
        mesh=mesh,
        compiler_params=pltpu.CompilerParams(needs_layout_passes=False),
        scratch_types=[
            pltpu.VMEM((7 * TBL,), jnp.float32),
            pltpu.VMEM((ELEMS_PER_TILE,), jnp.float32),
            pltpu.VMEM((ELEMS_PER_TILE,), jnp.float32),
            pltpu.VMEM((ELEMS_PER_TILE,), jnp.float32),
        ],
    )(_sc_body)
    return f(x, tab_flat)


def kernel(inputs, unnormalized_widths, unnormalized_heights,
           unnormalized_derivatives):
    tab = _build_tables(unnormalized_widths, unnormalized_heights,
                        unnormalized_derivatives)
    out, det = _sc_spline(inputs.reshape(-1), tab.reshape(-1))
    return (out.reshape(BATCH, V), det.reshape(BATCH, V))


# speedup vs baseline: 384.2999x; 1.1228x over previous
import functools

import jax
import jax.numpy as jnp
from jax import lax
from jax.experimental import pallas as pl
from jax.experimental.pallas import tpu as pltpu
from jax.experimental.pallas import tpu_sc as plsc

BATCH = 32768
V = 32
NUM_BINS = 30
MIN_BIN_W = 0.001
MIN_BIN_H = 0.001
MIN_DERIV = 0.001
_EDGE_CONST = 0.5392745158594121

NC = 2
NS = 16
NW = NC * NS
ROWS_PER_TILE = BATCH // NW

TBL = 1024
OFF_CB, OFF_CW, OFF_WI, OFF_CH, OFF_H, OFF_D, OFF_DL = (k * TBL for k in range(7))

LN2 = 0.6931471805599453
_LOG2_C = (-0.0251232, 0.27003746, -1.2479625, 3.24946656, -5.30170911,
           6.08989576, -3.03460285)


def _tables_tc_kernel(uw_ref, uh_ref, ud_ref, out_ref):
    uw = uw_ref[...]
    uh = uh_ref[...]
    ud = ud_ref[...]

    bi = lax.broadcasted_iota(jnp.int32, (NUM_BINS, V), 0)
    ji = lax.broadcasted_iota(jnp.int32, (NUM_BINS, V), 1)
    m = (bi < ji).astype(jnp.float32)

    def cumparams(u, min_size):
        mx = jnp.max(u, axis=-1, keepdims=True)
        e = jnp.exp(u - mx)
        p = e / jnp.sum(e, axis=-1, keepdims=True)
        p = min_size + (1.0 - min_size * NUM_BINS) * p
        cum = jnp.dot(p, m, preferred_element_type=jnp.float32,
                      precision=lax.Precision.HIGHEST)
        return cum

    cumw = cumparams(uw, MIN_BIN_W)
    cumh = cumparams(uh, MIN_BIN_H)
    w30 = cumw[:, 1:31] - cumw[:, 0:30]
    h30 = cumh[:, 1:31] - cumh[:, 0:30]
    ones2 = jnp.ones((V, 2), jnp.float32)

    col = lax.broadcasted_iota(jnp.int32, (V, V), 1)
    cb = jnp.where(col == 30, cumw + 1e-6, cumw)
    cb = jnp.where(col == 31, 1e30, cb)

    winv = jnp.concatenate([1.0 / w30, ones2], axis=1)
    hpad = jnp.concatenate([h30, ones2], axis=1)
    dlpad = jnp.concatenate([h30 / w30, ones2], axis=1)

    edge = jnp.full((V, 1), _EDGE_CONST, jnp.float32)
    dp31 = jnp.concatenate([edge, ud, edge], axis=1)
    sp = jnp.maximum(dp31, 0.0) + jnp.log(1.0 + jnp.exp(-jnp.abs(dp31)))
    dpad = jnp.concatenate([MIN_DERIV + sp, jnp.ones((V, 1), jnp.float32)],
                           axis=1)

    out_ref[0] = cb
    out_ref[1] = cumw
    out_ref[2] = winv
    out_ref[3] = cumh
    out_ref[4] = hpad
    out_ref[5] = dpad
    out_ref[6] = dlpad


def _build_tables(uw, uh, ud):
    return pl.pallas_call(
        _tables_tc_kernel,
        out_shape=jax.ShapeDtypeStruct((7, V, V), jnp.float32),
    )(uw, uh, ud)


def _log2_poly(x):
    i = lax.bitcast_convert_type(x, jnp.int32)
    e = lax.convert_element_type(
        lax.shift_right_arithmetic(i, 23) - 127, jnp.float32)
    mb = lax.bitwise_or(lax.bitwise_and(i, 0x007FFFFF), 0x3F800000)
    mm = lax.bitcast_convert_type(mb, jnp.float32)
    p = jnp.full((16,), _LOG2_C[0], jnp.float32)
    for c in _LOG2_C[1:]:
        p = p * mm + c
    return e + p


ELEMS_PER_TILE = ROWS_PER_TILE * V


def _sc_body(x_hbm, tab_hbm, out_hbm, det_hbm, tab_v, xin_v, out_v, det_v):
    wid = lax.axis_index("s") * NC + lax.axis_index("c")
    e0 = wid * ELEMS_PER_TILE
    pltpu.sync_copy(tab_hbm, tab_v)
    pltpu.sync_copy(x_hbm.at[pl.ds(e0, ELEMS_PER_TILE)], xin_v)

    lanes = lax.iota(jnp.int32, 16)
    vb0 = lanes * V
    vb1 = vb0 + 16 * V

    def do_vec(off, vbase):
        x = xin_v[pl.ds(off, 16)]
        b = jnp.zeros((16,), jnp.int32)
        for k in (16, 8, 4, 2, 1):
            t = b + k
            cbv = plsc.load_gather(tab_v, [vbase + t])
            b = jnp.where(cbv <= x, t, b)
        binr = jnp.minimum(b, NUM_BINS - 1)
        gi = vbase + binr
        cw = plsc.load_gather(tab_v, [gi + OFF_CW])
        winv = plsc.load_gather(tab_v, [gi + OFF_WI])
        ch = plsc.load_gather(tab_v, [gi + OFF_CH])
        h = plsc.load_gather(tab_v, [gi + OFF_H])
        d = plsc.load_gather(tab_v, [gi + OFF_D])
        dp = plsc.load_gather(tab_v, [gi + (OFF_D + 1)])
        dl = plsc.load_gather(tab_v, [gi + OFF_DL])

        th = (x - cw) * winv
        th2 = th * th
        th1 = th - th2
        num = h * (dl * th2 + d * th1)
        den = dl + (d + dp - 2.0 * dl) * th1
        spl = ch + num / den
        omt = 1.0 - th
        nd = (dl * dl) * (dp * th2 + 2.0 * dl * th1 + d * (omt * omt))
        logdet = LN2 * (_log2_poly(nd) - 2.0 * _log2_poly(den))
        inside = jnp.logical_and(x >= 0.0, x <= 1.0)
        out_v[pl.ds(off, 16)] = jnp.where(inside, spl, x)
        det_v[pl.ds(off, 16)] = jnp.where(
            inside, logdet, jnp.zeros((16,), jnp.float32))

    @plsc.parallel_loop(0, ELEMS_PER_TILE // 64, unroll=4)
    def _loop(i):
        base = i * 64
        do_vec(base, vb0)
        do_vec(base + 16, vb1)
        do_vec(base + 32, vb0)
        do_vec(base + 48, vb1)

    pltpu.sync_copy(out_v, out_hbm.at[pl.ds(e0, ELEMS_PER_TILE)])
    pltpu.sync_copy(det_v, det_hbm.at[pl.ds(e0, ELEMS_PER_TILE)])


@functools.partial(jax.jit)
def _sc_spline(x, tab_flat):
    mesh = plsc.VectorSubcoreMesh(core_axis_name="c", subcore_axis_name="s")
    f = functools.partial(
        pl.kernel,
        out_type=[
            jax.ShapeDtypeStruct((BATCH * V,), jnp.float32),
            jax.ShapeDtypeStruct((BATCH * V,), jnp.float32),
        ],
        mesh=mesh,
        compiler_params=pltpu.CompilerParams(needs_layout_passes=False),
        scratch_types=[
            pltpu.VMEM((7 * TBL,), jnp.float32),
            pltpu.VMEM((ELEMS_PER_TILE,), jnp.float32),
            pltpu.VMEM((ELEMS_PER_TILE,), jnp.float32),
            pltpu.VMEM((ELEMS_PER_TILE,), jnp.float32),
        ],
    )(_sc_body)
    return f(x, tab_flat)


def kernel(inputs, unnormalized_widths, unnormalized_heights,
           unnormalized_derivatives):
    tab = _build_tables(unnormalized_widths, unnormalized_heights,
                        unnormalized_derivatives)
    out, det = _sc_spline(inputs.reshape(-1), tab.reshape(-1))
    return (out.reshape(BATCH, V), det.reshape(BATCH, V))


# speedup vs baseline: 443.3163x; 1.1536x over previous
import functools

import jax
import jax.numpy as jnp
from jax import lax
from jax.experimental import pallas as pl
from jax.experimental.pallas import tpu as pltpu
from jax.experimental.pallas import tpu_sc as plsc

BATCH = 32768
V = 32
NUM_BINS = 30
MIN_BIN_W = 0.001
MIN_BIN_H = 0.001
MIN_DERIV = 0.001
_EDGE_CONST = 0.5392745158594121

NC = 2
NS = 16
NW = NC * NS
ROWS_PER_TILE = BATCH // NW
ELEMS_PER_TILE = ROWS_PER_TILE * V

LUT = 512
TBL = 1024
OFF_CB, OFF_CW, OFF_WI, OFF_CH, OFF_H, OFF_D = (k * TBL for k in range(6))

LN2 = 0.6931471805599453
_LOG2_C = (-0.0251232, 0.27003746, -1.2479625, 3.24946656, -5.30170911,
           6.08989576, -3.03460285)


def _tables_tc_kernel(uw_ref, uh_ref, ud_ref, tab_ref, lut_ref):
    uw = uw_ref[...]
    uh = uh_ref[...]
    ud = ud_ref[...]

    bi = lax.broadcasted_iota(jnp.int32, (NUM_BINS, V), 0)
    ji = lax.broadcasted_iota(jnp.int32, (NUM_BINS, V), 1)
    m = (bi < ji).astype(jnp.float32)

    def cumparams(u, min_size):
        mx = jnp.max(u, axis=-1, keepdims=True)
        e = jnp.exp(u - mx)
        p = e / jnp.sum(e, axis=-1, keepdims=True)
        p = min_size + (1.0 - min_size * NUM_BINS) * p
        cum = jnp.dot(p, m, preferred_element_type=jnp.float32,
                      precision=lax.Precision.HIGHEST)
        return cum

    cumw = cumparams(uw, MIN_BIN_W)
    cumh = cumparams(uh, MIN_BIN_H)
    w30 = cumw[:, 1:31] - cumw[:, 0:30]
    h30 = cumh[:, 1:31] - cumh[:, 0:30]
    ones2 = jnp.ones((V, 2), jnp.float32)

    col = lax.broadcasted_iota(jnp.int32, (V, V), 1)
    cb = jnp.where(col == 30, cumw + 1e-6, cumw)
    cb = jnp.where(col == 31, 1e30, cb)

    winv = jnp.concatenate([1.0 / w30, ones2], axis=1)
    hpad = jnp.concatenate([h30, ones2], axis=1)

    edge = jnp.full((V, 1), _EDGE_CONST, jnp.float32)
    dp31 = jnp.concatenate([edge, ud, edge], axis=1)
    sp = jnp.maximum(dp31, 0.0) + jnp.log(1.0 + jnp.exp(-jnp.abs(dp31)))
    dpad = jnp.concatenate([MIN_DERIV + sp, jnp.ones((V, 1), jnp.float32)],
                           axis=1)

    tab_ref[0] = cb
    tab_ref[1] = cumw
    tab_ref[2] = winv
    tab_ref[3] = cumh
    tab_ref[4] = hpad
    tab_ref[5] = dpad

    cells = (lax.broadcasted_iota(jnp.int32, (V, LUT), 1).astype(jnp.float32)
             * jnp.float32(1.0 / LUT))
    cnt = jnp.zeros((V, LUT), jnp.float32)
    for b in range(31):
        cnt = cnt + (cb[:, b:b + 1] <= cells).astype(jnp.float32)
    lut_ref[...] = jnp.clip(cnt.astype(jnp.int32) - 1, 0, NUM_BINS - 1)


def _build_tables(uw, uh, ud):
    return pl.pallas_call(
        _tables_tc_kernel,
        out_shape=(
            jax.ShapeDtypeStruct((6, V, V), jnp.float32),
            jax.ShapeDtypeStruct((V, LUT), jnp.int32),
        ),
    )(uw, uh, ud)


def _log2_poly(x):
    i = lax.bitcast_convert_type(x, jnp.int32)
    e = lax.convert_element_type(
        lax.shift_right_arithmetic(i, 23) - 127, jnp.float32)
    mb = lax.bitwise_or(lax.bitwise_and(i, 0x007FFFFF), 0x3F800000)
    mm = lax.bitcast_convert_type(mb, jnp.float32)
    p = jnp.full((16,), _LOG2_C[0], jnp.float32)
    for c in _LOG2_C[1:]:
        p = p * mm + c
    return e + p


def _sc_body(x_hbm, tab_hbm, lut_hbm, out_hbm, det_hbm,
             tab_v, lut_v, xin_v, out_v, det_v):
    wid = lax.axis_index("s") * NC + lax.axis_index("c")
    e0 = wid * ELEMS_PER_TILE
    pltpu.sync_copy(tab_hbm, tab_v)
    pltpu.sync_copy(lut_hbm, lut_v)
    pltpu.sync_copy(x_hbm.at[pl.ds(e0, ELEMS_PER_TILE)], xin_v)

    lanes = lax.iota(jnp.int32, 16)
    vb0 = lanes * V
    vb1 = vb0 + 16 * V
    vl0 = lanes * LUT
    vl1 = vl0 + 16 * LUT

    def do_vec(off, vbase, vlut):
        x = xin_v[pl.ds(off, 16)]
        cell = jnp.clip(
            lax.convert_element_type(x * jnp.float32(LUT), jnp.int32),
            0, LUT - 1)
        l = plsc.load_gather(lut_v, [vlut + cell])
        c1 = plsc.load_gather(tab_v, [vbase + (l + (OFF_CB + 1))])
        c2 = plsc.load_gather(tab_v, [vbase + (l + (OFF_CB + 2))])
        binr = l + (jnp.where(c1 <= x, 1, 0) + jnp.where(c2 <= x, 1, 0))
        binr = jnp.minimum(binr, NUM_BINS - 1)
        gi = vbase + binr
        cw = plsc.load_gather(tab_v, [gi + OFF_CW])
        winv = plsc.load_gather(tab_v, [gi + OFF_WI])
        ch = plsc.load_gather(tab_v, [gi + OFF_CH])
        h = plsc.load_gather(tab_v, [gi + OFF_H])
        d = plsc.load_gather(tab_v, [gi + OFF_D])
        dp = plsc.load_gather(tab_v, [gi + (OFF_D + 1)])
        dl = h * winv

        th = (x - cw) * winv
        th2 = th * th
        th1 = th - th2
        num = h * (dl * th2 + d * th1)
        den = dl + (d + dp - 2.0 * dl) * th1
        rcp = 1.0 / den
        spl = ch + num * rcp
        omt = 1.0 - th
        nd = (dl * dl) * (dp * th2 + 2.0 * dl * th1 + d * (omt * omt))
        logdet = LN2 * _log2_poly(nd * (rcp * rcp))
        inside = jnp.logical_and(x >= 0.0, x <= 1.0)
        out_v[pl.ds(off, 16)] = jnp.where(inside, spl, x)
        det_v[pl.ds(off, 16)] = jnp.where(
            inside, logdet, jnp.zeros((16,), jnp.float32))

    @plsc.parallel_loop(0, ELEMS_PER_TILE // 64, unroll=4)
    def _loop(i):
        base = i * 64
        do_vec(base, vb0, vl0)
        do_vec(base + 16, vb1, vl1)
        do_vec(base + 32, vb0, vl0)
        do_vec(base + 48, vb1, vl1)

    pltpu.sync_copy(out_v, out_hbm.at[pl.ds(e0, ELEMS_PER_TILE)])
    pltpu.sync_copy(det_v, det_hbm.at[pl.ds(e0, ELEMS_PER_TILE)])


@functools.partial(jax.jit)
def _sc_spline(x, tab_flat, lut_flat):
    mesh = plsc.VectorSubcoreMesh(core_axis_name="c", subcore_axis_name="s")
    f = functools.partial(
        pl.kernel,
        out_type=[
            jax.ShapeDtypeStruct((BATCH * V,), jnp.float32),
            jax.ShapeDtypeStruct((BATCH * V,), jnp.float32),
        ],
        mesh=mesh,
        compiler_params=pltpu.CompilerParams(needs_layout_passes=False),
        scratch_types=[
            pltpu.VMEM((6 * TBL,), jnp.float32),
            pltpu.VMEM((V * LUT,), jnp.int32),
            pltpu.VMEM((ELEMS_PER_TILE,), jnp.float32),
            pltpu.VMEM((ELEMS_PER_TILE,), jnp.float32),
            pltpu.VMEM((ELEMS_PER_TILE,), jnp.float32),
        ],
    )(_sc_body)
    return f(x, tab_flat, lut_flat)


def kernel(inputs, unnormalized_widths, unnormalized_heights,
           unnormalized_derivatives):
    tab, lut = _build_tables(unnormalized_widths, unnormalized_heights,
                             unnormalized_derivatives)
    out, det = _sc_spline(inputs.reshape(-1), tab.reshape(-1),
                          lut.reshape(-1))
    return (out.reshape(BATCH, V), det.reshape(BATCH, V))


# speedup vs baseline: 1477.2812x; 3.3323x over previous
import functools

import jax
import jax.numpy as jnp
from jax import lax
from jax.experimental import pallas as pl
from jax.experimental.pallas import tpu as pltpu
from jax.experimental.pallas import tpu_sc as plsc

BATCH = 32768
V = 32
NUM_BINS = 30
MIN_BIN_W = 0.001
MIN_BIN_H = 0.001
MIN_DERIV = 0.001
_EDGE_CONST = 0.5392745158594121

NC = 2
NS = 16
NW = NC * NS
ROWS_PER_TILE = BATCH // NW
ELEMS_PER_TILE = ROWS_PER_TILE * V

LUT = 512
TBL = 1024
OFF_CB, OFF_CW, OFF_WI, OFF_CH, OFF_H, OFF_D = (k * TBL for k in range(6))

LN2 = 0.6931471805599453
_LOG2_C = (-0.0251232, 0.27003746, -1.2479625, 3.24946656, -5.30170911,
           6.08989576, -3.03460285)


def _tables_tc_kernel(uw_ref, uh_ref, ud_ref, tab_ref, lut_ref):
    uw = uw_ref[...]
    uh = uh_ref[...]
    ud = ud_ref[...]

    bi = lax.broadcasted_iota(jnp.int32, (NUM_BINS, V), 0)
    ji = lax.broadcasted_iota(jnp.int32, (NUM_BINS, V), 1)
    m = (bi < ji).astype(jnp.float32)

    def cumparams(u, min_size):
        mx = jnp.max(u, axis=-1, keepdims=True)
        e = jnp.exp(u - mx)
        p = e / jnp.sum(e, axis=-1, keepdims=True)
        p = min_size + (1.0 - min_size * NUM_BINS) * p
        cum = jnp.dot(p, m, preferred_element_type=jnp.float32,
                      precision=lax.Precision.HIGHEST)
        return cum

    cumw = cumparams(uw, MIN_BIN_W)
    cumh = cumparams(uh, MIN_BIN_H)
    w30 = cumw[:, 1:31] - cumw[:, 0:30]
    h30 = cumh[:, 1:31] - cumh[:, 0:30]
    ones2 = jnp.ones((V, 2), jnp.float32)

    col = lax.broadcasted_iota(jnp.int32, (V, V), 1)
    cb = jnp.where(col == 30, cumw + 1e-6, cumw)
    cb = jnp.where(col == 31, 1e30, cb)

    winv = jnp.concatenate([1.0 / w30, ones2], axis=1)
    hpad = jnp.concatenate([h30, ones2], axis=1)

    edge = jnp.full((V, 1), _EDGE_CONST, jnp.float32)
    dp31 = jnp.concatenate([edge, ud, edge], axis=1)
    sp = jnp.maximum(dp31, 0.0) + jnp.log(1.0 + jnp.exp(-jnp.abs(dp31)))
    dpad = jnp.concatenate([MIN_DERIV + sp, jnp.ones((V, 1), jnp.float32)],
                           axis=1)

    tab_ref[0] = cb
    tab_ref[1] = cumw
    tab_ref[2] = winv
    tab_ref[3] = cumh
    tab_ref[4] = hpad
    tab_ref[5] = dpad

    cells = (lax.broadcasted_iota(jnp.int32, (V, LUT), 1).astype(jnp.float32)
             * jnp.float32(1.0 / LUT))
    cnt = jnp.zeros((V, LUT), jnp.float32)
    for b in range(31):
        cnt = cnt + (cb[:, b:b + 1] <= cells).astype(jnp.float32)
    lut_ref[...] = jnp.clip(cnt.astype(jnp.int32) - 1, 0, NUM_BINS - 1)


def _build_tables(uw, uh, ud):
    return pl.pallas_call(
        _tables_tc_kernel,
        out_shape=(
            jax.ShapeDtypeStruct((6, V, V), jnp.float32),
            jax.ShapeDtypeStruct((V, LUT), jnp.int32),
        ),
    )(uw, uh, ud)


def _log2_poly(x):
    i = lax.bitcast_convert_type(x, jnp.int32)
    e = lax.convert_element_type(
        lax.shift_right_arithmetic(i, 23) - 127, jnp.float32)
    mb = lax.bitwise_or(lax.bitwise_and(i, 0x007FFFFF), 0x3F800000)
    mm = lax.bitcast_convert_type(mb, jnp.float32)
    p = jnp.full((16,), _LOG2_C[0], jnp.float32)
    for c in _LOG2_C[1:]:
        p = p * mm + c
    return e + p


COLS_PER_TILE = BATCH // NW


def _sc_body(x_hbm, tab_hbm, lut_hbm, out_hbm, det_hbm,
             tab_v, lut_v, xin_v, out_v, det_v):
    wid = lax.axis_index("s") * NC + lax.axis_index("c")
    c0 = wid * COLS_PER_TILE
    pltpu.sync_copy(tab_hbm, tab_v)
    pltpu.sync_copy(lut_hbm, lut_v)
    pltpu.sync_copy(x_hbm.at[:, pl.ds(c0, COLS_PER_TILE)], xin_v)

    def do_vec(r, col, tb, lb):
        x = xin_v[r, pl.ds(col, 16)]
        cell = jnp.clip(
            lax.convert_element_type(x * jnp.float32(LUT), jnp.int32),
            0, LUT - 1)
        l = plsc.load_gather(lut_v, [cell + lb])
        c1 = plsc.load_gather(tab_v, [l + (tb + OFF_CB + 1)])
        c2 = plsc.load_gather(tab_v, [l + (tb + OFF_CB + 2)])
        binr = l + (jnp.where(c1 <= x, 1, 0) + jnp.where(c2 <= x, 1, 0))
        binr = jnp.minimum(binr, NUM_BINS - 1)
        gi = binr + tb
        cw = plsc.load_gather(tab_v, [gi + OFF_CW])
        winv = plsc.load_gather(tab_v, [gi + OFF_WI])
        ch = plsc.load_gather(tab_v, [gi + OFF_CH])
        h = plsc.load_gather(tab_v, [gi + OFF_H])
        d = plsc.load_gather(tab_v, [gi + OFF_D])
        dp = plsc.load_gather(tab_v, [gi + (OFF_D + 1)])
        dl = h * winv

        th = (x - cw) * winv
        th2 = th * th
        th1 = th - th2
        num = h * (dl * th2 + d * th1)
        den = dl + (d + dp - 2.0 * dl) * th1
        rcp = 1.0 / den
        spl = ch + num * rcp
        omt = 1.0 - th
        nd = (dl * dl) * (dp * th2 + 2.0 * dl * th1 + d * (omt * omt))
        logdet = LN2 * _log2_poly(nd * (rcp * rcp))
        inside = jnp.logical_and(x >= 0.0, x <= 1.0)
        out_v[r, pl.ds(col, 16)] = jnp.where(inside, spl, x)
        det_v[r, pl.ds(col, 16)] = jnp.where(
            inside, logdet, jnp.zeros((16,), jnp.float32))

    @plsc.parallel_loop(0, V * COLS_PER_TILE // 16, unroll=4)
    def _loop(i):
        r = lax.shift_right_logical(i, 6)
        col = lax.bitwise_and(i, 63) * 16
        do_vec(r, col, r * V, r * LUT)

    pltpu.sync_copy(out_v, out_hbm.at[:, pl.ds(c0, COLS_PER_TILE)])
    pltpu.sync_copy(det_v, det_hbm.at[:, pl.ds(c0, COLS_PER_TILE)])


@functools.partial(jax.jit)
def _sc_spline(xt, tab_flat, lut_flat):
    mesh = plsc.VectorSubcoreMesh(core_axis_name="c", subcore_axis_name="s")
    f = functools.partial(
        pl.kernel,
        out_type=[
            jax.ShapeDtypeStruct((V, BATCH), jnp.float32),
            jax.ShapeDtypeStruct((V, BATCH), jnp.float32),
        ],
        mesh=mesh,
        compiler_params=pltpu.CompilerParams(needs_layout_passes=False),
        scratch_types=[
            pltpu.VMEM((6 * TBL,), jnp.float32),
            pltpu.VMEM((V * LUT,), jnp.int32),
            pltpu.VMEM((V, COLS_PER_TILE), jnp.float32),
            pltpu.VMEM((V, COLS_PER_TILE), jnp.float32),
            pltpu.VMEM((V, COLS_PER_TILE), jnp.float32),
        ],
    )(_sc_body)
    return f(xt, tab_flat, lut_flat)


def kernel(inputs, unnormalized_widths, unnormalized_heights,
           unnormalized_derivatives):
    tab, lut = _build_tables(unnormalized_widths, unnormalized_heights,
                             unnormalized_derivatives)
    out_t, det_t = _sc_spline(inputs.T, tab.reshape(-1), lut.reshape(-1))
    return (out_t.T, det_t.T)
